# Initial kernel scaffold; baseline (speedup 1.0000x reference)
import jax
import jax.numpy as jnp
from jax.experimental import pallas as pl


def kernel(x, edge_index, batch, n_vertices, gt_vertices, params):
    raise NotImplementedError("write your pallas kernel here")


# speedup vs baseline: 1.1865x; 1.1865x over previous
import functools

import jax
import jax.numpy as jnp
import numpy as np
from jax.experimental import pallas as pl
from jax.experimental.pallas import tpu as pltpu

_F32 = jnp.float32


def _mm(a, w):
    return jax.lax.dot_general(a, w, (((1,), (1,)), ((), ())),
                               preferred_element_type=_F32)


def _mmn(a, b):
    return jax.lax.dot_general(a, b, (((1,), (0,)), ((), ())),
                               preferred_element_type=_F32)


def _gelu(x):
    return 0.5 * x * (1.0 + jax.lax.erf(x * np.float32(0.7071067811865476)))


def _ln(x, g, b):
    m = jnp.mean(x, axis=-1, keepdims=True)
    v = jnp.mean((x - m) ** 2, axis=-1, keepdims=True)
    return (x - m) / jnp.sqrt(v + 1e-5) * g + b


def _l1_body(a_ref, x_ref, wl_ref, bl_ref, wr_ref, g_ref, b_ref,
             h_ref, rc_ref):
    a = a_ref[...]
    cnt = a[:, 15:16]
    rc = 1.0 / jnp.maximum(cnt, 1.0)
    am = a * rc
    y = _mm(am, wl_ref[...]) + bl_ref[...] + _mm(x_ref[...], wr_ref[...])
    h_ref[...] = _gelu(_ln(y, g_ref[...], b_ref[...]))
    rc_ref[...] = rc


def _post_body(a_ref, rc_ref, h_ref, wr_ref, bl_ref, g_ref, b_ref, o_ref):
    y = a_ref[...] * rc_ref[...] + bl_ref[...] + _mm(h_ref[...], wr_ref[...])
    o_ref[...] = _gelu(_ln(y, g_ref[...], b_ref[...]))


def _post4_body(a_ref, rc_ref, h_ref, wr_ref, bl_ref, g_ref, b_ref, z_ref):
    y = a_ref[...] * rc_ref[...] + bl_ref[...] + _mm(h_ref[...], wr_ref[...])
    hh = _gelu(_ln(y, g_ref[...], b_ref[...]))
    nb, d = hh.shape
    z_ref[...] = jnp.mean(hh.reshape(nb // 128, 128, d), axis=1)


def _proj_body(h_ref, w_ref, p_ref):
    p_ref[...] = _mm(h_ref[...], w_ref[...])


_KB = 512


def _vqa_body(z_ref, c_ref, ws_ref, cw_ref, idx_ref, bm_ref, bi_ref):
    j = pl.program_id(0)
    bsz, emb = z_ref.shape
    kb = c_ref.shape[0]
    cw = _mm(c_ref[...], ws_ref[...])
    cw_ref[...] = cw
    cw2 = jnp.sum(cw * cw, axis=1, keepdims=True)
    za = jnp.concatenate(
        [z_ref[...], jnp.ones((bsz, 1), _F32),
         jnp.zeros((bsz, emb - 1), _F32)], axis=1)
    cwa = jnp.concatenate(
        [-2.0 * cw, cw2, jnp.zeros((kb, emb - 1), _F32)], axis=1)
    s = _mm(za, cwa)
    m = jnp.min(s, axis=1, keepdims=True)
    iota = jax.lax.broadcasted_iota(jnp.int32, s.shape, 1) + j * _KB
    ib = jnp.min(jnp.where(s <= m, iota, jnp.int32(2 ** 30)), axis=1,
                 keepdims=True)
    prevm = jnp.where(j == 0, jnp.full_like(m, np.float32(3e38)), bm_ref[...])
    previ = jnp.where(j == 0, jnp.zeros_like(ib), bi_ref[...])
    take = m < prevm
    bm_ref[...] = jnp.where(take, m, prevm)
    bi_ref[...] = jnp.where(take, ib, previ)

    @pl.when(j == pl.num_programs(0) - 1)
    def _():
        idx_ref[...] = bi_ref[...]


def _vqb_body(z_ref, idx_ref, cw_ref, wv_ref, bv_ref, wo_ref, bo_ref,
              q_ref, vo_ref, com_ref, qacc_ref):
    j = pl.program_id(0)
    bsz = z_ref.shape[0]
    iota = jax.lax.broadcasted_iota(jnp.int32, (bsz, _KB), 1) + j * _KB
    oh = (iota == idx_ref[...]).astype(_F32)
    part = _mmn(oh, cw_ref[...])
    prev = jnp.where(j == 0, jnp.zeros_like(part), qacc_ref[...])
    qacc_ref[...] = prev + part

    @pl.when(j == pl.num_programs(0) - 1)
    def _():
        z = z_ref[...]
        q = qacc_ref[...]
        q_ref[...] = q
        d = z - q
        com_ref[...] = (jnp.sum(d * d) / np.float32(z.size)).reshape(1, 1)
        vo_ref[...] = (_mm(_mm(q, wv_ref[...]) + bv_ref[...], wo_ref[...])
                       + bo_ref[...])


def _dec_body(vo_ref, nv_ref, gt_ref, vq_ref, ang_ref, anb_ref,
              w1_ref, b1_ref, w2_ref, b2_ref, w3_ref, b3_ref,
              rec_ref, loss_ref, acc_ref):
    pb = vo_ref.shape[0]
    mv = vq_ref.shape[0]
    vo = vo_ref[...]
    x = vo[:, None, :] + vq_ref[...][None, :, :]
    x = x.reshape(pb * mv, x.shape[-1])
    hd = _ln(x, ang_ref[...], anb_ref[...])
    hd = _gelu(_mm(hd, w1_ref[...]) + b1_ref[...])
    hd = _gelu(_mm(hd, w2_ref[...]) + b2_ref[...])
    co = _mm(hd, w3_ref[...]) + b3_ref[...]
    nv = nv_ref[...]
    nvrep = jnp.broadcast_to(nv.reshape(pb, 1, 1), (pb, mv, 1)
                             ).reshape(pb * mv, 1)
    rowi = jax.lax.broadcasted_iota(jnp.int32, (pb * mv, 1), 0)
    rowmask = ((rowi % mv) < nvrep).astype(_F32)
    co = co * rowmask
    rec_ref[...] = co.reshape(pb, mv, 8)
    g8 = gt_ref[...]
    big = np.float32(1e10)
    tot = jnp.zeros((), _F32)
    for p in range(pb):
        r = co[p * mv:(p + 1) * mv]
        g = g8[p]
        g2 = jnp.sum(g * g, axis=1, keepdims=True)
        ga = jnp.concatenate([g, g2, jnp.zeros((mv, 7), _F32)], axis=1)
        ra = jnp.concatenate([-2.0 * r, jnp.ones((mv, 1), _F32),
                              jnp.zeros((mv, 7), _F32)], axis=1)
        r2 = jnp.sum(r * r, axis=1, keepdims=True)
        dd = r2 + _mm(ra, ga)
        nv_p = nv[p:p + 1, :]
        mlane = (jax.lax.broadcasted_iota(jnp.int32, (1, mv), 1)
                 < nv_p).astype(_F32)
        msub = (jax.lax.broadcasted_iota(jnp.int32, (mv, 1), 0)
                < nv_p).astype(_F32)
        mrg = jnp.min(jnp.where(mlane > 0, dd, big), axis=1, keepdims=True)
        mgr = jnp.min(jnp.where(msub > 0, dd, big), axis=0, keepdims=True)
        cv = jnp.maximum(jnp.sum(mlane), 1.0)
        tot = tot + (jnp.sum(mrg * msub) + jnp.sum(mgr * mlane)) / cv
    step = pl.program_id(0)
    prev = jnp.where(step == 0, jnp.zeros((1, 1), _F32), acc_ref[...])
    acc_ref[...] = prev + tot.reshape(1, 1)
    nb_total = pl.num_programs(0)

    @pl.when(step == nb_total - 1)
    def _():
        loss_ref[...] = acc_ref[...] / np.float32(nb_total * pb)


_NBLK = 512


def _full(shape):
    return pl.BlockSpec(shape, lambda *a: tuple(0 for _ in shape))


def _rows(shape):
    return pl.BlockSpec(shape, lambda i: (i,) + tuple(0 for _ in shape[1:]))


def _l1_call(a16, x16, wl, bl, wr, g, b):
    n = a16.shape[0]
    grid = (n // _NBLK,)
    return pl.pallas_call(
        _l1_body,
        grid=grid,
        in_specs=[_rows((_NBLK, 16)), _rows((_NBLK, 16)),
                  _full(wl.shape), _full(bl.shape), _full(wr.shape),
                  _full(g.shape), _full(b.shape)],
        out_specs=[_rows((_NBLK, wl.shape[0])), _rows((_NBLK, 1))],
        out_shape=[jax.ShapeDtypeStruct((n, wl.shape[0]), _F32),
                   jax.ShapeDtypeStruct((n, 1), _F32)],
    )(a16, x16, wl, bl, wr, g, b)


def _post_call(a, rc, h, wr, bl, g, b):
    n, dout = a.shape
    grid = (n // _NBLK,)
    return pl.pallas_call(
        _post_body,
        grid=grid,
        in_specs=[_rows((_NBLK, dout)), _rows((_NBLK, 1)),
                  _rows((_NBLK, h.shape[1])), _full(wr.shape),
                  _full(bl.shape), _full(g.shape), _full(b.shape)],
        out_specs=_rows((_NBLK, dout)),
        out_shape=jax.ShapeDtypeStruct((n, dout), _F32),
    )(a, rc, h, wr, bl, g, b)


def _post4_call(a, rc, h, wr, bl, g, b):
    n, dout = a.shape
    nblk = 1024
    grid = (n // nblk,)
    npatch = nblk // 128
    return pl.pallas_call(
        _post4_body,
        grid=grid,
        in_specs=[_rows((nblk, dout)), _rows((nblk, 1)),
                  _rows((nblk, h.shape[1])), _full(wr.shape),
                  _full(bl.shape), _full(g.shape), _full(b.shape)],
        out_specs=_rows((npatch, dout)),
        out_shape=jax.ShapeDtypeStruct((n // 128, dout), _F32),
    )(a, rc, h, wr, bl, g, b)


def _proj_call(h, w):
    n = h.shape[0]
    grid = (n // _NBLK,)
    return pl.pallas_call(
        _proj_body,
        grid=grid,
        in_specs=[_rows((_NBLK, h.shape[1])), _full(w.shape)],
        out_specs=_rows((_NBLK, w.shape[0])),
        out_shape=jax.ShapeDtypeStruct((n, w.shape[0]), _F32),
    )(h, w)


def _vq_call(z, c, ws, wv, bv, wo, bo):
    bsz, emb = z.shape
    k = c.shape[0]
    grid = (k // _KB,)
    cw, idx2 = pl.pallas_call(
        _vqa_body,
        grid=grid,
        in_specs=[_full(z.shape), _rows((_KB, emb)), _full(ws.shape)],
        out_specs=[_rows((_KB, emb)), _full((bsz, 1))],
        out_shape=[jax.ShapeDtypeStruct((k, emb), _F32),
                   jax.ShapeDtypeStruct((bsz, 1), jnp.int32)],
        scratch_shapes=[pltpu.VMEM((bsz, 1), _F32),
                        pltpu.VMEM((bsz, 1), jnp.int32)],
    )(z, c, ws)
    quant, vo, com = pl.pallas_call(
        _vqb_body,
        grid=grid,
        in_specs=[_full(z.shape), _full((bsz, 1)), _rows((_KB, emb)),
                  _full(wv.shape), _full(bv.shape), _full(wo.shape),
                  _full(bo.shape)],
        out_specs=[_full((bsz, emb)), _full((bsz, emb)), _full((1, 1))],
        out_shape=[jax.ShapeDtypeStruct((bsz, emb), _F32),
                   jax.ShapeDtypeStruct((bsz, emb), _F32),
                   jax.ShapeDtypeStruct((1, 1), _F32)],
        scratch_shapes=[pltpu.VMEM((bsz, emb), _F32)],
    )(z, idx2, cw, wv, bv, wo, bo)
    return idx2, quant, vo, com


def _dec_call(vo, nv, gt8, vq, ang, anb, w1, b1, w2, b2, w3, b3):
    bsz, emb = vo.shape
    mv = vq.shape[0]
    pb = 8
    grid = (bsz // pb,)
    return pl.pallas_call(
        _dec_body,
        grid=grid,
        in_specs=[_rows((pb, emb)), _rows((pb, 1)),
                  _rows((pb, mv, 8)), _full(vq.shape),
                  _full(ang.shape), _full(anb.shape),
                  _full(w1.shape), _full(b1.shape),
                  _full(w2.shape), _full(b2.shape),
                  _full(w3.shape), _full(b3.shape)],
        out_specs=[_rows((pb, mv, 8)), _full((1, 1))],
        out_shape=[jax.ShapeDtypeStruct((bsz, mv, 8), _F32),
                   jax.ShapeDtypeStruct((1, 1), _F32)],
        scratch_shapes=[pltpu.VMEM((1, 1), _F32)],
    )(vo, nv, gt8, vq, ang, anb, w1, b1, w2, b2, w3, b3)


def _aggregate(p_feat, src, dst, n):
    return jax.ops.segment_sum(p_feat[src], dst, num_segments=n)


def kernel(x, edge_index, batch, n_vertices, gt_vertices, params):
    p = params
    n = x.shape[0]
    bsz, maxv = gt_vertices.shape[0], gt_vertices.shape[1]
    src, dst = edge_index[0], edge_index[1]

    x16 = jnp.concatenate([x, jnp.ones((n, 1), _F32)], axis=1)
    pad16 = lambda w: jnp.pad(w, ((0, 0), (0, 1)))
    row = lambda v: v[None, :]

    a1 = _aggregate(x16, src, dst, n)
    h1, rc = _l1_call(a1, x16, pad16(p['c1_Wl']), row(p['c1_bl']),
                      pad16(p['c1_Wr']), row(p['n1_g']), row(p['n1_b']))

    p2 = _proj_call(h1, p['c2_Wl'])
    a2 = _aggregate(p2, src, dst, n)
    h2 = _post_call(a2, rc, h1, p['c2_Wr'], row(p['c2_bl']),
                    row(p['n2_g']), row(p['n2_b']))

    p3 = _proj_call(h2, p['c3_Wl'])
    a3 = _aggregate(p3, src, dst, n)
    h3 = _post_call(a3, rc, h2, p['c3_Wr'], row(p['c3_bl']),
                    row(p['n3_g']), row(p['n3_b']))

    p4 = _proj_call(h3, p['c4_Wl'])
    a4 = _aggregate(p4, src, dst, n)
    z = _post4_call(a4, rc, h3, p['c4_Wr'], row(p['c4_bl']),
                    row(p['n4_g']), row(p['n4_b']))

    idx2, quant, vo, com = _vq_call(z, p['C'], p['Wsim'], p['Wv'],
                                    row(p['bv']), p['Wo'], row(p['bo']))

    w3p = jnp.pad(p['m3_W'], ((0, 5), (0, 0)))
    b3p = jnp.pad(p['m3_b'], (0, 5))
    gt8 = jnp.pad(gt_vertices, ((0, 0), (0, 0), (0, 5)))
    rec8, loss = _dec_call(vo, n_vertices.reshape(bsz, 1).astype(jnp.int32),
                           gt8, p['vq'], row(p['an_g']), row(p['an_b']),
                           p['m1_W'], row(p['m1_b']),
                           p['m2_W'], row(p['m2_b']), w3p, row(b3p))

    recon = rec8[:, :, :3]
    recon_loss = loss[0, 0]
    commit = com[0, 0]
    embed = commit
    total = recon_loss + commit + embed
    idx = idx2[:, 0]
    return recon, total, recon_loss, commit, embed, idx, z


# speedup vs baseline: 1.5251x; 1.2854x over previous
import functools

import jax
import jax.numpy as jnp
import numpy as np
from jax import lax
from jax.experimental import pallas as pl
from jax.experimental.pallas import tpu as pltpu
from jax.experimental.pallas import tpu_sc as plsc

_F32 = jnp.float32


_BF16 = jnp.bfloat16


def _mm(a, w):
    return jax.lax.dot_general(a.astype(_BF16), w.astype(_BF16),
                               (((1,), (1,)), ((), ())),
                               preferred_element_type=_F32)


def _mmh(a, w):
    return jax.lax.dot_general(a, w, (((1,), (1,)), ((), ())),
                               preferred_element_type=_F32,
                               precision=jax.lax.Precision.HIGHEST)


def _mmnh(a, b):
    return jax.lax.dot_general(a, b, (((1,), (0,)), ((), ())),
                               preferred_element_type=_F32,
                               precision=jax.lax.Precision.HIGHEST)


def _gelu(x):
    return 0.5 * x * (1.0 + jax.lax.erf(x * np.float32(0.7071067811865476)))


def _ln(x, g, b):
    m = jnp.mean(x, axis=-1, keepdims=True)
    v = jnp.mean((x - m) ** 2, axis=-1, keepdims=True)
    return (x - m) / jnp.sqrt(v + 1e-5) * g + b


def _l1_body(a_ref, x_ref, wl_ref, bl_ref, wr_ref, g_ref, b_ref,
             h_ref, cm_ref):
    a = a_ref[...]
    cm = jnp.maximum(a[:, 15:16], 1.0)
    am = a / cm
    y = _mm(am, wl_ref[...]) + bl_ref[...] + _mm(x_ref[...], wr_ref[...])
    h_ref[...] = _gelu(_ln(y, g_ref[...], b_ref[...]))
    cm_ref[...] = cm


def _post_body(a_ref, cm_ref, h_ref, wl_ref, wr_ref, bl_ref, g_ref, b_ref,
               o_ref):
    am = a_ref[...] / cm_ref[...]
    y = _mm(am, wl_ref[...]) + bl_ref[...] + _mm(h_ref[...], wr_ref[...])
    o_ref[...] = _gelu(_ln(y, g_ref[...], b_ref[...]))


def _post4_body(a_ref, cm_ref, h_ref, wl_ref, wr_ref, bl_ref, g_ref, b_ref,
                z_ref):
    am = a_ref[...] / cm_ref[...]
    y = _mm(am, wl_ref[...]) + bl_ref[...] + _mm(h_ref[...], wr_ref[...])
    hh = _gelu(_ln(y, g_ref[...], b_ref[...]))
    nb, d = hh.shape
    z_ref[...] = jnp.mean(hh.reshape(nb // 128, 128, d), axis=1)


_KB = 512


def _vqa_body(z_ref, c_ref, ws_ref, cw_ref, idx_ref, bm_ref, bi_ref):
    j = pl.program_id(0)
    bsz, emb = z_ref.shape
    kb = c_ref.shape[0]
    cw = _mm(c_ref[...], ws_ref[...])
    cw_ref[...] = cw
    cw2 = jnp.sum(cw * cw, axis=1, keepdims=True)
    ones8 = jnp.concatenate(
        [jnp.ones((bsz, 1), _F32), jnp.zeros((bsz, 7), _F32)], axis=1)
    c2m = jnp.concatenate([cw2, jnp.zeros((kb, 7), _F32)], axis=1)
    s = _mmh(ones8, c2m) - 2.0 * _mm(z_ref[...], cw)
    m = jnp.min(s, axis=1, keepdims=True)
    iota = jax.lax.broadcasted_iota(jnp.int32, s.shape, 1) + j * _KB
    ib = jnp.min(jnp.where(s <= m, iota, jnp.int32(2 ** 30)), axis=1,
                 keepdims=True)
    prevm = jnp.where(j == 0, jnp.full_like(m, np.float32(3e38)), bm_ref[...])
    previ = jnp.where(j == 0, jnp.zeros_like(ib), bi_ref[...])
    take = m < prevm
    bm_ref[...] = jnp.where(take, m, prevm)
    bi_ref[...] = jnp.where(take, ib, previ)

    @pl.when(j == pl.num_programs(0) - 1)
    def _():
        idx_ref[...] = bi_ref[...]


def _vqb_body(z_ref, idx_ref, cw_ref, wv_ref, bv_ref, wo_ref, bo_ref,
              q_ref, vo_ref, com_ref, qacc_ref):
    j = pl.program_id(0)
    bsz = z_ref.shape[0]
    iota = jax.lax.broadcasted_iota(jnp.int32, (bsz, _KB), 1) + j * _KB
    oh = (iota == idx_ref[...]).astype(_F32)
    part = _mmnh(oh, cw_ref[...])
    prev = jnp.where(j == 0, jnp.zeros_like(part), qacc_ref[...])
    qacc_ref[...] = prev + part

    @pl.when(j == pl.num_programs(0) - 1)
    def _():
        z = z_ref[...]
        q = qacc_ref[...]
        q_ref[...] = q
        d = z - q
        com_ref[...] = (jnp.sum(d * d) / np.float32(z.size)).reshape(1, 1)
        vo_ref[...] = (_mm(_mm(q, wv_ref[...]) + bv_ref[...], wo_ref[...])
                       + bo_ref[...])


def _dec_body(vo_ref, nv_ref, gt_ref, vq_ref, ang_ref, anb_ref,
              w1_ref, b1_ref, w2_ref, b2_ref, w3_ref, b3_ref,
              rec_ref, loss_ref, acc_ref):
    pb = vo_ref.shape[0]
    mv = vq_ref.shape[0]
    vo = vo_ref[...]
    x = vo[:, None, :] + vq_ref[...][None, :, :]
    x = x.reshape(pb * mv, x.shape[-1])
    hd = _ln(x, ang_ref[...], anb_ref[...])
    hd = _gelu(_mm(hd, w1_ref[...]) + b1_ref[...])
    hd = _gelu(_mm(hd, w2_ref[...]) + b2_ref[...])
    co = _mm(hd, w3_ref[...]) + b3_ref[...]
    nv = nv_ref[...]
    nvrep = jnp.broadcast_to(nv.reshape(pb, 1, 1), (pb, mv, 1)
                             ).reshape(pb * mv, 1)
    rowi = jax.lax.broadcasted_iota(jnp.int32, (pb * mv, 1), 0)
    rowmask = ((rowi % mv) < nvrep).astype(_F32)
    co = co * rowmask
    rec_ref[...] = co.reshape(pb, mv, 8)
    g8 = gt_ref[...]
    big = np.float32(1e10)
    tot = jnp.zeros((), _F32)
    for p in range(pb):
        r = co[p * mv:(p + 1) * mv]
        g = g8[p]
        g2 = jnp.sum(g * g, axis=1, keepdims=True)
        ga = jnp.concatenate([g, g2, jnp.zeros((mv, 7), _F32)], axis=1)
        ra = jnp.concatenate([-2.0 * r, jnp.ones((mv, 1), _F32),
                              jnp.zeros((mv, 7), _F32)], axis=1)
        r2 = jnp.sum(r * r, axis=1, keepdims=True)
        dd = r2 + _mmh(ra, ga)
        nv_p = nv[p:p + 1, :]
        mlane = (jax.lax.broadcasted_iota(jnp.int32, (1, mv), 1)
                 < nv_p).astype(_F32)
        msub = (jax.lax.broadcasted_iota(jnp.int32, (mv, 1), 0)
                < nv_p).astype(_F32)
        mrg = jnp.min(jnp.where(mlane > 0, dd, big), axis=1, keepdims=True)
        mgr = jnp.min(jnp.where(msub > 0, dd, big), axis=0, keepdims=True)
        cv = jnp.maximum(jnp.sum(mlane), 1.0)
        tot = tot + (jnp.sum(mrg * msub) + jnp.sum(mgr * mlane)) / cv
    step = pl.program_id(0)
    prev = jnp.where(step == 0, jnp.zeros((1, 1), _F32), acc_ref[...])
    acc_ref[...] = prev + tot.reshape(1, 1)
    nb_total = pl.num_programs(0)

    @pl.when(step == nb_total - 1)
    def _():
        loss_ref[...] = acc_ref[...] / np.float32(nb_total * pb)


_NBLK = 512


def _full(shape):
    return pl.BlockSpec(shape, lambda *a: tuple(0 for _ in shape))


def _rows(shape):
    return pl.BlockSpec(shape, lambda i: (i,) + tuple(0 for _ in shape[1:]))


def _l1_call(a128, x16, wl, bl, wr, g, b):
    n = a128.shape[0]
    grid = (n // _NBLK,)
    return pl.pallas_call(
        _l1_body,
        grid=grid,
        in_specs=[_rows((_NBLK, a128.shape[1])), _rows((_NBLK, 16)),
                  _full(wl.shape), _full(bl.shape), _full(wr.shape),
                  _full(g.shape), _full(b.shape)],
        out_specs=[_rows((_NBLK, wl.shape[0])), _rows((_NBLK, 1))],
        out_shape=[jax.ShapeDtypeStruct((n, wl.shape[0]), _F32),
                   jax.ShapeDtypeStruct((n, 1), _F32)],
    )(a128, x16, wl, bl, wr, g, b)


def _post_call(a, cm, h, wl, wr, bl, g, b):
    n, din = a.shape
    dout = wl.shape[0]
    grid = (n // _NBLK,)
    return pl.pallas_call(
        _post_body,
        grid=grid,
        in_specs=[_rows((_NBLK, din)), _rows((_NBLK, 1)),
                  _rows((_NBLK, h.shape[1])), _full(wl.shape),
                  _full(wr.shape), _full(bl.shape), _full(g.shape),
                  _full(b.shape)],
        out_specs=_rows((_NBLK, dout)),
        out_shape=jax.ShapeDtypeStruct((n, dout), _F32),
    )(a, cm, h, wl, wr, bl, g, b)


def _post4_call(a, cm, h, wl, wr, bl, g, b):
    n, din = a.shape
    dout = wl.shape[0]
    nblk = 1024
    grid = (n // nblk,)
    npatch = nblk // 128
    return pl.pallas_call(
        _post4_body,
        grid=grid,
        in_specs=[_rows((nblk, din)), _rows((nblk, 1)),
                  _rows((nblk, h.shape[1])), _full(wl.shape),
                  _full(wr.shape), _full(bl.shape), _full(g.shape),
                  _full(b.shape)],
        out_specs=_rows((npatch, dout)),
        out_shape=jax.ShapeDtypeStruct((n // 128, dout), _F32),
    )(a, cm, h, wl, wr, bl, g, b)


def _vq_call(z, c, ws, wv, bv, wo, bo):
    bsz, emb = z.shape
    k = c.shape[0]
    grid = (k // _KB,)
    cw, idx2 = pl.pallas_call(
        _vqa_body,
        grid=grid,
        in_specs=[_full(z.shape), _rows((_KB, emb)), _full(ws.shape)],
        out_specs=[_rows((_KB, emb)), _full((bsz, 1))],
        out_shape=[jax.ShapeDtypeStruct((k, emb), _F32),
                   jax.ShapeDtypeStruct((bsz, 1), jnp.int32)],
        scratch_shapes=[pltpu.VMEM((bsz, 1), _F32),
                        pltpu.VMEM((bsz, 1), jnp.int32)],
    )(z, c, ws)
    quant, vo, com = pl.pallas_call(
        _vqb_body,
        grid=grid,
        in_specs=[_full(z.shape), _full((bsz, 1)), _rows((_KB, emb)),
                  _full(wv.shape), _full(bv.shape), _full(wo.shape),
                  _full(bo.shape)],
        out_specs=[_full((bsz, emb)), _full((bsz, emb)), _full((1, 1))],
        out_shape=[jax.ShapeDtypeStruct((bsz, emb), _F32),
                   jax.ShapeDtypeStruct((bsz, emb), _F32),
                   jax.ShapeDtypeStruct((1, 1), _F32)],
        scratch_shapes=[pltpu.VMEM((bsz, emb), _F32)],
    )(z, idx2, cw, wv, bv, wo, bo)
    return idx2, quant, vo, com


def _dec_call(vo, nv, gt8, vq, ang, anb, w1, b1, w2, b2, w3, b3):
    bsz, emb = vo.shape
    mv = vq.shape[0]
    pb = 8
    grid = (bsz // pb,)
    return pl.pallas_call(
        _dec_body,
        grid=grid,
        in_specs=[_rows((pb, emb)), _rows((pb, 1)),
                  _rows((pb, mv, 8)), _full(vq.shape),
                  _full(ang.shape), _full(anb.shape),
                  _full(w1.shape), _full(b1.shape),
                  _full(w2.shape), _full(b2.shape),
                  _full(w3.shape), _full(b3.shape)],
        out_specs=[_rows((pb, mv, 8)), _full((1, 1))],
        out_shape=[jax.ShapeDtypeStruct((bsz, mv, 8), _F32),
                   jax.ShapeDtypeStruct((1, 1), _F32)],
        scratch_shapes=[pltpu.VMEM((1, 1), _F32)],
    )(vo, nv, gt8, vq, ang, anb, w1, b1, w2, b2, w3, b3)


_EC = 128


def _sc_agg(p_feat, srcs, dsts, starts, ncks, n, d, bs):
    nit = (n // bs) // 32
    mesh = plsc.VectorSubcoreMesh(core_axis_name="c", subcore_axis_name="s")

    @functools.partial(
        pl.kernel, mesh=mesh,
        out_type=jax.ShapeDtypeStruct((n, d), _F32),
        scratch_types=[
            pltpu.VMEM((_EC,), jnp.int32),
            pltpu.VMEM((_EC,), jnp.int32),
            pltpu.VMEM((16,), jnp.int32),
            pltpu.VMEM((16,), jnp.int32),
            pltpu.VMEM((_EC, d), _F32),
            pltpu.VMEM((bs + 8, d), _F32),
            pltpu.SMEM((16,), jnp.int32),
            pltpu.SMEM((16,), jnp.int32),
            pltpu.SemaphoreType.DMA,
        ],
    )
    def agg(p_hbm, srcs_hbm, dsts_hbm, st_hbm, nc_hbm, a_hbm,
            idxb, dstb, stv, ncv, rows, acc, sst, snc, sem):
        c = lax.axis_index("c")
        s = lax.axis_index("s")
        w = s * 2 + c
        rowbase = pl.multiple_of(s * 32 + c * 16, 16)
        pltpu.sync_copy(st_hbm.at[pl.ds(rowbase, 16)], stv)
        pltpu.sync_copy(nc_hbm.at[pl.ds(rowbase, 16)], ncv)
        stval = stv[...]
        ncval = ncv[...]
        for i in range(nit):
            sst[i] = stval[i]
            snc[i] = ncval[i]
        zero = jnp.zeros((16,), _F32)

        def it_body(it, carry):
            n0 = (it * 32 + w) * bs
            myst = sst[it]
            mync = snc[it]

            def zrow(rr, cz):
                for cc in range(d // 16):
                    acc[rr, pl.ds(cc * 16, 16)] = zero
                return cz

            lax.fori_loop(0, bs, zrow, 0)

            def chunk(k, ck):
                base = pl.multiple_of(myst + k * _EC, _EC)
                pltpu.sync_copy(srcs_hbm.at[pl.ds(base, _EC)], idxb)
                pltpu.sync_copy(dsts_hbm.at[pl.ds(base, _EC)], dstb)
                pltpu.async_copy(p_hbm.at[idxb], rows, sem).wait()

                def grp(gi, cg):
                    dv = dstb[pl.ds(gi * 16, 16)]
                    rel = dv - n0
                    ok = (rel >= 0) & (rel < bs)
                    relc = jnp.where(ok, rel, bs)
                    e0 = gi * 16
                    for j in range(16):
                        rj = relc[j]
                        ej = e0 + j
                        for cc in range(d // 16):
                            sl = pl.ds(cc * 16, 16)
                            acc[rj, sl] = acc[rj, sl] + rows[ej, sl]
                    return cg

                lax.fori_loop(0, _EC // 16, grp, 0)
                return ck

            lax.fori_loop(0, mync, chunk, 0)
            nw = pl.multiple_of(n0, bs)
            pltpu.sync_copy(acc.at[pl.ds(0, bs)], a_hbm.at[pl.ds(nw, bs)])
            return carry

        lax.fori_loop(0, nit, it_body, 0)

    return agg(p_feat, srcs, dsts, starts, ncks)


def _edge_plan(dsts_sorted, n):
    nb = n // (n // 512)
    bs = n // 512
    roff = jnp.searchsorted(dsts_sorted, jnp.arange(nb + 1) * bs
                            ).astype(jnp.int32)
    bt = roff[:-1] & ~(_EC - 1)
    et = ((roff[1:] + _EC - 1) // _EC) * _EC
    nck = (et - bt) // _EC
    nit = nb // 32
    it = jnp.arange(nit)[None, None, :]
    s_i = jnp.arange(16)[:, None, None]
    c_i = jnp.arange(2)[None, :, None]
    gmap = it * 32 + s_i * 2 + c_i
    st = bt[gmap].reshape(-1).astype(jnp.int32)
    nc = nck[gmap].reshape(-1).astype(jnp.int32)
    return st, nc


def kernel(x, edge_index, batch, n_vertices, gt_vertices, params):
    p = params
    n = x.shape[0]
    e = edge_index.shape[1]
    bsz, maxv = gt_vertices.shape[0], gt_vertices.shape[1]
    src, dst = edge_index[0], edge_index[1]

    order = jnp.argsort(dst)
    srcs = jnp.concatenate([src[order].astype(jnp.int32),
                            jnp.zeros((_EC,), jnp.int32)])
    dsts_s = dst[order].astype(jnp.int32)
    dsts = jnp.concatenate([dsts_s, jnp.full((_EC,), n, jnp.int32)])
    bs = n // 512
    stp, ncp = _edge_plan(dsts_s, n)

    x16 = jnp.concatenate([x, jnp.ones((n, 1), _F32)], axis=1)
    x128 = jnp.pad(x16, ((0, 0), (0, 112)))
    pad16 = lambda w: jnp.pad(w, ((0, 0), (0, 1)))
    pad128 = lambda w: jnp.pad(w, ((0, 0), (0, 113)))
    row = lambda v: v[None, :]

    a1 = _sc_agg(x128, srcs, dsts, stp, ncp, n, 128, bs)
    h1, cm = _l1_call(a1, x16, pad128(p['c1_Wl']), row(p['c1_bl']),
                      pad16(p['c1_Wr']), row(p['n1_g']), row(p['n1_b']))

    a2 = _sc_agg(h1, srcs, dsts, stp, ncp, n, 256, bs)
    h2 = _post_call(a2, cm, h1, p['c2_Wl'], p['c2_Wr'], row(p['c2_bl']),
                    row(p['n2_g']), row(p['n2_b']))

    a3 = _sc_agg(h2, srcs, dsts, stp, ncp, n, 256, bs)
    h3 = _post_call(a3, cm, h2, p['c3_Wl'], p['c3_Wr'], row(p['c3_bl']),
                    row(p['n3_g']), row(p['n3_b']))

    a4 = _sc_agg(h3, srcs, dsts, stp, ncp, n, 256, bs)
    z = _post4_call(a4, cm, h3, p['c4_Wl'], p['c4_Wr'], row(p['c4_bl']),
                    row(p['n4_g']), row(p['n4_b']))

    idx2, quant, vo, com = _vq_call(z, p['C'], p['Wsim'], p['Wv'],
                                    row(p['bv']), p['Wo'], row(p['bo']))

    w3p = jnp.pad(p['m3_W'], ((0, 5), (0, 0)))
    b3p = jnp.pad(p['m3_b'], (0, 5))
    gt8 = jnp.pad(gt_vertices, ((0, 0), (0, 0), (0, 5)))
    rec8, loss = _dec_call(vo, n_vertices.reshape(bsz, 1).astype(jnp.int32),
                           gt8, p['vq'], row(p['an_g']), row(p['an_b']),
                           p['m1_W'], row(p['m1_b']),
                           p['m2_W'], row(p['m2_b']), w3p, row(b3p))

    recon = rec8[:, :, :3]
    recon_loss = loss[0, 0]
    commit = com[0, 0]
    embed = commit
    total = recon_loss + commit + embed
    idx = idx2[:, 0]
    return recon, total, recon_loss, commit, embed, idx, z
